# baseline (speedup 1.0000x reference)
import functools

import jax
import jax.numpy as jnp
from jax import lax
from jax.experimental import pallas as pl
from jax.experimental.pallas import tpu as pltpu
from jax.experimental.pallas import tpu_sc as plsc

N = 50000
E = 800000
F_IN = 14
H = 256

NC = 2
NS = 16
CHUNK = 256
IBLK = 14 * CHUNK
EPAD = ((E + NC * NS * CHUNK - 1) // (NC * NS * CHUNK)) * (NC * NS * CHUNK)
ACC_ROWS = 50048
RT = ACC_ROWS // NS
CR = 128
CR_TAIL = RT - (RT // CR) * CR
CRO = 256
CRO_TAIL = RT - (RT // CRO) * CRO
NSLAB = 8
SLAB = H // NSLAB

_SELU_ALPHA = 1.6732632423543772
_SELU_SCALE = 1.0507009873554805


def _selu(x):
    return _SELU_SCALE * jnp.where(x > 0, x, _SELU_ALPHA * (jnp.exp(x) - 1.0))


def _sc_segsum16_body(x16_hbm, src_hbm, dst2_hbm, z16_hbm, part_hbm,
                      acc_sh, src_blk, dst_blk, rows_v, buf_v, sg, ss):
    c = lax.axis_index("c")
    t = lax.axis_index("s")
    tile_edges = EPAD // (NC * NS)
    base = (c * NS + t) * tile_edges

    pltpu.sync_copy(z16_hbm, buf_v)
    r0 = t * RT

    def zero_body(i, _):
        pltpu.sync_copy(buf_v, acc_sh.at[pl.ds(r0 + i * CR, CR)])
        return 0
    lax.fori_loop(0, RT // CR, zero_body, 0)
    pltpu.sync_copy(buf_v.at[pl.ds(0, CR_TAIL)],
                    acc_sh.at[pl.ds(r0 + (RT // CR) * CR, CR_TAIL)])
    plsc.subcore_barrier()

    def blk_body(g, _):
        e0 = base + g * IBLK
        pltpu.sync_copy(src_hbm.at[pl.ds(e0, IBLK)], src_blk)
        pltpu.sync_copy(dst2_hbm.at[pl.ds(e0 // CHUNK, IBLK // CHUNK)],
                        dst_blk)

        def edge_body(k, _):
            for b in range(2):
                ch = 2 * k + b

                @pl.when(g + k > 0)
                def _():
                    pltpu.make_async_copy(rows_v[b],
                                          acc_sh.at[dst_blk.at[ch]],
                                          ss[b]).wait()
                pltpu.async_copy(
                    x16_hbm.at[src_blk.at[pl.ds(ch * CHUNK, CHUNK)]],
                    rows_v[b], sg[b])
            for b in range(2):
                ch = 2 * k + b
                pltpu.make_async_copy(
                    x16_hbm.at[src_blk.at[pl.ds(ch * CHUNK, CHUNK)]],
                    rows_v[b], sg[b]).wait()
                pltpu.async_copy(rows_v[b], acc_sh.at[dst_blk.at[ch]],
                                 ss[b], add=True)
            return 0
        lax.fori_loop(0, IBLK // (2 * CHUNK), edge_body, 0)
        return 0
    lax.fori_loop(0, tile_edges // IBLK, blk_body, 0)
    for b in range(2):
        pltpu.make_async_copy(rows_v[b], acc_sh.at[dst_blk.at[0]],
                              ss[b]).wait()
    plsc.subcore_barrier()

    def out_body(i, _):
        rr = r0 + i * CR
        pltpu.sync_copy(acc_sh.at[pl.ds(rr, CR)], buf_v)
        pltpu.sync_copy(buf_v, part_hbm.at[c, pl.ds(rr, CR)])
        return 0
    lax.fori_loop(0, RT // CR, out_body, 0)
    rr = r0 + (RT // CR) * CR
    pltpu.sync_copy(acc_sh.at[pl.ds(rr, CR_TAIL)], buf_v.at[pl.ds(0, CR_TAIL)])
    pltpu.sync_copy(buf_v.at[pl.ds(0, CR_TAIL)],
                    part_hbm.at[c, pl.ds(rr, CR_TAIL)])


_sc_segsum16 = functools.partial(
    pl.kernel,
    out_type=jax.ShapeDtypeStruct((NC, ACC_ROWS, 16), jnp.float32),
    mesh=plsc.VectorSubcoreMesh(core_axis_name="c", subcore_axis_name="s"),
    compiler_params=pltpu.CompilerParams(use_tc_tiling_on_sc=False),
    scratch_types=[
        pltpu.VMEM_SHARED((ACC_ROWS, 16), jnp.float32),
        pltpu.VMEM((IBLK,), jnp.int32),
        pltpu.VMEM((IBLK // CHUNK, CHUNK), jnp.int32),
        [pltpu.VMEM((CHUNK, 16), jnp.float32) for _ in range(2)],
        pltpu.VMEM((CR, 16), jnp.float32),
        [pltpu.SemaphoreType.DMA for _ in range(2)],
        [pltpu.SemaphoreType.DMA for _ in range(2)],
    ],
)(_sc_segsum16_body)


def _sc_segsum256_body(p2_hbm, src_hbm, dst2_hbm, z32_hbm, r2_hbm, out_hbm,
                       acc_sh, src_blk, dst_blk, rows_v, buf_v, sg, ss):
    c = lax.axis_index("c")
    t = lax.axis_index("s")
    tile_edges = EPAD // NS
    base = t * tile_edges
    r0 = t * RT

    for j in range(NSLAB // NC):
        s = c * (NSLAB // NC) + j
        s_off = s * N

        pltpu.sync_copy(z32_hbm, buf_v)

        def zero_body(i, _):
            pltpu.sync_copy(buf_v, acc_sh.at[pl.ds(r0 + i * CR, CR)])
            return 0
        lax.fori_loop(0, RT // CR, zero_body, 0)
        pltpu.sync_copy(buf_v.at[pl.ds(0, CR_TAIL)],
                        acc_sh.at[pl.ds(r0 + (RT // CR) * CR, CR_TAIL)])
        plsc.subcore_barrier()

        p2s = p2_hbm.at[pl.ds(s_off, N)]

        def blk_body(g, _):
            e0 = base + g * IBLK
            pltpu.sync_copy(src_hbm.at[pl.ds(e0, IBLK)], src_blk)
            pltpu.sync_copy(dst2_hbm.at[pl.ds(e0 // CHUNK, IBLK // CHUNK)],
                            dst_blk)

            def edge_body(k, _):
                for b in range(2):
                    ch = 2 * k + b

                    @pl.when(g + k > 0)
                    def _():
                        pltpu.make_async_copy(rows_v[b],
                                              acc_sh.at[dst_blk.at[ch]],
                                              ss[b]).wait()
                    pltpu.async_copy(
                        p2s.at[src_blk.at[pl.ds(ch * CHUNK, CHUNK)]],
                        rows_v[b], sg[b])
                for b in range(2):
                    ch = 2 * k + b
                    pltpu.make_async_copy(
                        p2s.at[src_blk.at[pl.ds(ch * CHUNK, CHUNK)]],
                        rows_v[b], sg[b]).wait()
                    pltpu.async_copy(rows_v[b], acc_sh.at[dst_blk.at[ch]],
                                     ss[b], add=True)
                return 0
            lax.fori_loop(0, IBLK // (2 * CHUNK), edge_body, 0)
            return 0
        lax.fori_loop(0, tile_edges // IBLK, blk_body, 0)
        for b in range(2):
            pltpu.make_async_copy(rows_v[b], acc_sh.at[dst_blk.at[0]],
                                  ss[b]).wait()
        plsc.subcore_barrier()

        cs = s * SLAB

        def selu_chunk(nrows):
            def row_body(i, _):
                for half in range(SLAB // 16):
                    ix = (i, pl.ds(half * 16, 16))
                    a = rows_v[0][ix] + rows_v[1][ix]
                    e = _SELU_ALPHA * (jnp.exp(a) - 1.0)
                    rows_v[0][ix] = _SELU_SCALE * jnp.where(a > 0, a, e)
                return 0
            lax.fori_loop(0, nrows, row_body, 0)

        def out_body(i, _):
            rr = r0 + i * CRO
            pltpu.sync_copy(acc_sh.at[pl.ds(rr, CRO)], rows_v[0])
            pltpu.sync_copy(r2_hbm.at[pl.ds(rr, CRO), pl.ds(cs, SLAB)],
                            rows_v[1])
            selu_chunk(CRO)
            pltpu.sync_copy(rows_v[0],
                            out_hbm.at[pl.ds(rr, CRO), pl.ds(cs, SLAB)])
            return 0
        lax.fori_loop(0, RT // CRO, out_body, 0)

        rr = r0 + (RT // CRO) * CRO
        pltpu.sync_copy(acc_sh.at[pl.ds(rr, CRO_TAIL)],
                        rows_v[0].at[pl.ds(0, CRO_TAIL)])
        pltpu.sync_copy(r2_hbm.at[pl.ds(rr, 8), pl.ds(cs, SLAB)],
                        rows_v[1].at[pl.ds(0, 8)])

        @pl.when(t < NS - 1)
        def _():
            pltpu.sync_copy(
                r2_hbm.at[pl.ds(rr + 8, CRO_TAIL - 8), pl.ds(cs, SLAB)],
                rows_v[1].at[pl.ds(8, CRO_TAIL - 8)])
        selu_chunk(CRO_TAIL)
        pltpu.sync_copy(rows_v[0].at[pl.ds(0, 8)],
                        out_hbm.at[pl.ds(rr, 8), pl.ds(cs, SLAB)])

        @pl.when(t < NS - 1)
        def _():
            pltpu.sync_copy(rows_v[0].at[pl.ds(8, CRO_TAIL - 8)],
                            out_hbm.at[pl.ds(rr + 8, CRO_TAIL - 8),
                                       pl.ds(cs, SLAB)])
        plsc.subcore_barrier()


_sc_segsum256 = functools.partial(
    pl.kernel,
    out_type=jax.ShapeDtypeStruct((N, H), jnp.float32),
    mesh=plsc.VectorSubcoreMesh(core_axis_name="c", subcore_axis_name="s"),
    compiler_params=pltpu.CompilerParams(use_tc_tiling_on_sc=False),
    scratch_types=[
        pltpu.VMEM_SHARED((ACC_ROWS, SLAB), jnp.float32),
        pltpu.VMEM((IBLK,), jnp.int32),
        pltpu.VMEM((IBLK // CHUNK, CHUNK), jnp.int32),
        [pltpu.VMEM((CHUNK, SLAB), jnp.float32) for _ in range(2)],
        pltpu.VMEM((CR, SLAB), jnp.float32),
        [pltpu.SemaphoreType.DMA for _ in range(2)],
        [pltpu.SemaphoreType.DMA for _ in range(2)],
    ],
)(_sc_segsum256_body)


_BR = 1000


def _tc_dense_body(part_ref, x_ref, w1rel_ref, w1root_ref, b1_ref,
                   w2rel_ref, w2root_ref, b2_ref, p2_ref, r2_ref):
    agg = part_ref[0] + part_ref[1]
    pre = (jnp.dot(agg, w1rel_ref[...], preferred_element_type=jnp.float32)
           + jnp.dot(x_ref[...], w1root_ref[...],
                     preferred_element_type=jnp.float32)
           + b1_ref[...])
    h = _selu(pre)
    p2 = jnp.dot(h, w2rel_ref[...], preferred_element_type=jnp.float32)
    r2_ref[...] = (jnp.dot(h, w2root_ref[...],
                           preferred_element_type=jnp.float32) + b2_ref[...])
    for j in range(NSLAB):
        p2_ref[j] = p2[:, j * SLAB:(j + 1) * SLAB]


def _tc_dense(part, x, w1rel16, w1root, b1, w2rel, w2root, b2):
    return pl.pallas_call(
        _tc_dense_body,
        grid=(N // _BR,),
        in_specs=[
            pl.BlockSpec((NC, _BR, 16), lambda i: (0, i, 0)),
            pl.BlockSpec((_BR, F_IN), lambda i: (i, 0)),
            pl.BlockSpec((16, H), lambda i: (0, 0)),
            pl.BlockSpec((F_IN, H), lambda i: (0, 0)),
            pl.BlockSpec((1, H), lambda i: (0, 0)),
            pl.BlockSpec((H, H), lambda i: (0, 0)),
            pl.BlockSpec((H, H), lambda i: (0, 0)),
            pl.BlockSpec((1, H), lambda i: (0, 0)),
        ],
        out_specs=[
            pl.BlockSpec((NSLAB, _BR, SLAB), lambda i: (0, i, 0)),
            pl.BlockSpec((_BR, H), lambda i: (i, 0)),
        ],
        out_shape=[
            jax.ShapeDtypeStruct((NSLAB, N, SLAB), jnp.float32),
            jax.ShapeDtypeStruct((N, H), jnp.float32),
        ],
    )(part, x, w1rel16, w1root, b1, w2rel, w2root, b2)


def kernel(x, edge_index, batch, W1_rel, W1_root, b1, W2_rel, W2_root, b2):
    src = edge_index[0]
    dst = edge_index[1]
    npad = EPAD - E
    src_p = jnp.concatenate([src, jnp.zeros((npad,), jnp.int32)])
    dst_p = jnp.concatenate([dst, jnp.full((npad,), N, jnp.int32)])
    dst2 = dst_p.reshape(EPAD // CHUNK, CHUNK)

    x16 = jnp.pad(x, ((0, 0), (0, 16 - F_IN)))
    w1rel16 = jnp.pad(W1_rel, ((0, 16 - F_IN), (0, 0)))
    z16 = jnp.zeros((CR, 16), jnp.float32)
    z32 = jnp.zeros((CR, SLAB), jnp.float32)

    part = _sc_segsum16(x16, src_p, dst2, z16)
    p2, r2 = _tc_dense(part, x, w1rel16, W1_root, b1.reshape(1, H),
                       W2_rel, W2_root, b2.reshape(1, H))
    return _sc_segsum256(p2.reshape(NSLAB * N, SLAB), src_p, dst2, z32, r2)


# speedup vs baseline: 1.0085x; 1.0085x over previous
import functools

import jax
import jax.numpy as jnp
from jax import lax
from jax.experimental import pallas as pl
from jax.experimental.pallas import tpu as pltpu
from jax.experimental.pallas import tpu_sc as plsc

N = 50000
E = 800000
F_IN = 14
H = 256

NC = 2
NS = 16
CHUNK = 256
IBLK = 14 * CHUNK
EPAD = ((E + NC * NS * CHUNK - 1) // (NC * NS * CHUNK)) * (NC * NS * CHUNK)
ACC_ROWS = 50048
RT = ACC_ROWS // NS
CR = 128
CR_TAIL = RT - (RT // CR) * CR
CRO = 256
CRO_TAIL = RT - (RT // CRO) * CRO
NSLAB = 8
SLAB = H // NSLAB

_SELU_ALPHA = 1.6732632423543772
_SELU_SCALE = 1.0507009873554805


def _selu(x):
    return _SELU_SCALE * jnp.where(x > 0, x, _SELU_ALPHA * (jnp.exp(x) - 1.0))


def _sc_segsum16_body(x16_hbm, src_hbm, dst2_hbm, z16_hbm, part_hbm,
                      acc_sh, src_blk, dst_blk, rows_v, buf_v, sg, ss):
    c = lax.axis_index("c")
    t = lax.axis_index("s")
    tile_edges = EPAD // (NC * NS)
    base = (c * NS + t) * tile_edges

    pltpu.sync_copy(z16_hbm, buf_v)
    r0 = t * RT

    def zero_body(i, _):
        pltpu.sync_copy(buf_v, acc_sh.at[pl.ds(r0 + i * CR, CR)])
        return 0
    lax.fori_loop(0, RT // CR, zero_body, 0)
    pltpu.sync_copy(buf_v.at[pl.ds(0, CR_TAIL)],
                    acc_sh.at[pl.ds(r0 + (RT // CR) * CR, CR_TAIL)])
    plsc.subcore_barrier()

    def blk_body(g, _):
        e0 = base + g * IBLK
        pltpu.sync_copy(src_hbm.at[pl.ds(e0, IBLK)], src_blk)
        pltpu.sync_copy(dst2_hbm.at[pl.ds(e0 // CHUNK, IBLK // CHUNK)],
                        dst_blk)

        def edge_body(k, _):
            for b in range(2):
                ch = 2 * k + b

                @pl.when(g + k > 0)
                def _():
                    pltpu.make_async_copy(rows_v[b],
                                          acc_sh.at[dst_blk.at[ch]],
                                          ss[b]).wait()
                pltpu.async_copy(
                    x16_hbm.at[src_blk.at[pl.ds(ch * CHUNK, CHUNK)]],
                    rows_v[b], sg[b])
            for b in range(2):
                ch = 2 * k + b
                pltpu.make_async_copy(
                    x16_hbm.at[src_blk.at[pl.ds(ch * CHUNK, CHUNK)]],
                    rows_v[b], sg[b]).wait()
                pltpu.async_copy(rows_v[b], acc_sh.at[dst_blk.at[ch]],
                                 ss[b], add=True)
            return 0
        lax.fori_loop(0, IBLK // (2 * CHUNK), edge_body, 0)
        return 0
    lax.fori_loop(0, tile_edges // IBLK, blk_body, 0)
    for b in range(2):
        pltpu.make_async_copy(rows_v[b], acc_sh.at[dst_blk.at[0]],
                              ss[b]).wait()
    plsc.subcore_barrier()

    def out_body(i, _):
        rr = r0 + i * CR
        pltpu.sync_copy(acc_sh.at[pl.ds(rr, CR)], buf_v)
        pltpu.sync_copy(buf_v, part_hbm.at[c, pl.ds(rr, CR)])
        return 0
    lax.fori_loop(0, RT // CR, out_body, 0)
    rr = r0 + (RT // CR) * CR
    pltpu.sync_copy(acc_sh.at[pl.ds(rr, CR_TAIL)], buf_v.at[pl.ds(0, CR_TAIL)])
    pltpu.sync_copy(buf_v.at[pl.ds(0, CR_TAIL)],
                    part_hbm.at[c, pl.ds(rr, CR_TAIL)])


_sc_segsum16 = functools.partial(
    pl.kernel,
    out_type=jax.ShapeDtypeStruct((NC, ACC_ROWS, 16), jnp.float32),
    mesh=plsc.VectorSubcoreMesh(core_axis_name="c", subcore_axis_name="s"),
    compiler_params=pltpu.CompilerParams(use_tc_tiling_on_sc=False),
    scratch_types=[
        pltpu.VMEM_SHARED((ACC_ROWS, 16), jnp.float32),
        pltpu.VMEM((IBLK,), jnp.int32),
        pltpu.VMEM((IBLK // CHUNK, CHUNK), jnp.int32),
        [pltpu.VMEM((CHUNK, 16), jnp.float32) for _ in range(2)],
        pltpu.VMEM((CR, 16), jnp.float32),
        [pltpu.SemaphoreType.DMA for _ in range(2)],
        [pltpu.SemaphoreType.DMA for _ in range(2)],
    ],
)(_sc_segsum16_body)


def _sc_segsum256_body(p2_hbm, src_hbm, dst2_hbm, z32_hbm, r2_hbm, out_hbm,
                       acc_sh, src_blk, dst_blk, rows_v, buf_v, sg, ss, sz):
    c = lax.axis_index("c")
    t = lax.axis_index("s")
    tile_edges = EPAD // NS
    base = t * tile_edges
    r0 = t * RT

    for j in range(NSLAB // NC):
        s = c * (NSLAB // NC) + j
        s_off = s * N

        pltpu.sync_copy(z32_hbm, buf_v)

        def zero_start(i, _):
            pltpu.async_copy(buf_v, acc_sh.at[pl.ds(r0 + i * CR, CR)], sz)
            return 0
        lax.fori_loop(0, RT // CR, zero_start, 0)

        def zero_drain(i, _):
            pltpu.make_async_copy(buf_v,
                                  acc_sh.at[pl.ds(r0 + i * CR, CR)],
                                  sz).wait()
            return 0
        pltpu.sync_copy(buf_v.at[pl.ds(0, CR_TAIL)],
                        acc_sh.at[pl.ds(r0 + (RT // CR) * CR, CR_TAIL)])
        lax.fori_loop(0, RT // CR, zero_drain, 0)
        plsc.subcore_barrier()

        p2s = p2_hbm.at[pl.ds(s_off, N)]

        def blk_body(g, _):
            e0 = base + g * IBLK
            pltpu.sync_copy(src_hbm.at[pl.ds(e0, IBLK)], src_blk)
            pltpu.sync_copy(dst2_hbm.at[pl.ds(e0 // CHUNK, IBLK // CHUNK)],
                            dst_blk)

            def edge_body(k, _):
                for b in range(2):
                    ch = 2 * k + b

                    @pl.when(g + k > 0)
                    def _():
                        pltpu.make_async_copy(rows_v[b],
                                              acc_sh.at[dst_blk.at[ch]],
                                              ss[b]).wait()
                    pltpu.async_copy(
                        p2s.at[src_blk.at[pl.ds(ch * CHUNK, CHUNK)]],
                        rows_v[b], sg[b])
                for b in range(2):
                    ch = 2 * k + b
                    pltpu.make_async_copy(
                        p2s.at[src_blk.at[pl.ds(ch * CHUNK, CHUNK)]],
                        rows_v[b], sg[b]).wait()
                    pltpu.async_copy(rows_v[b], acc_sh.at[dst_blk.at[ch]],
                                     ss[b], add=True)
                return 0
            lax.fori_loop(0, IBLK // (2 * CHUNK), edge_body, 0)
            return 0
        lax.fori_loop(0, tile_edges // IBLK, blk_body, 0)
        for b in range(2):
            pltpu.make_async_copy(rows_v[b], acc_sh.at[dst_blk.at[0]],
                                  ss[b]).wait()
        plsc.subcore_barrier()

        cs = s * SLAB

        def selu_chunk(nrows):
            def row_body(i, _):
                for half in range(SLAB // 16):
                    ix = (i, pl.ds(half * 16, 16))
                    a = rows_v[0][ix] + rows_v[1][ix]
                    e = _SELU_ALPHA * (jnp.exp(a) - 1.0)
                    rows_v[0][ix] = _SELU_SCALE * jnp.where(a > 0, a, e)
                return 0
            lax.fori_loop(0, nrows, row_body, 0)

        def out_body(i, _):
            rr = r0 + i * CRO

            @pl.when(i > 0)
            def _():
                pltpu.make_async_copy(
                    rows_v[0],
                    out_hbm.at[pl.ds(rr - CRO, CRO), pl.ds(cs, SLAB)],
                    ss[0]).wait()
            pltpu.async_copy(acc_sh.at[pl.ds(rr, CRO)], rows_v[0], sg[0])
            pltpu.async_copy(r2_hbm.at[pl.ds(rr, CRO), pl.ds(cs, SLAB)],
                             rows_v[1], sg[1])
            pltpu.make_async_copy(acc_sh.at[pl.ds(rr, CRO)], rows_v[0],
                                  sg[0]).wait()
            pltpu.make_async_copy(r2_hbm.at[pl.ds(rr, CRO), pl.ds(cs, SLAB)],
                                  rows_v[1], sg[1]).wait()
            selu_chunk(CRO)
            pltpu.async_copy(rows_v[0],
                             out_hbm.at[pl.ds(rr, CRO), pl.ds(cs, SLAB)],
                             ss[0])
            return 0
        lax.fori_loop(0, RT // CRO, out_body, 0)
        rr_last = r0 + (RT // CRO - 1) * CRO
        pltpu.make_async_copy(rows_v[0],
                              out_hbm.at[pl.ds(rr_last, CRO),
                                         pl.ds(cs, SLAB)],
                              ss[0]).wait()

        rr = r0 + (RT // CRO) * CRO
        pltpu.sync_copy(acc_sh.at[pl.ds(rr, CRO_TAIL)],
                        rows_v[0].at[pl.ds(0, CRO_TAIL)])
        pltpu.sync_copy(r2_hbm.at[pl.ds(rr, 8), pl.ds(cs, SLAB)],
                        rows_v[1].at[pl.ds(0, 8)])

        @pl.when(t < NS - 1)
        def _():
            pltpu.sync_copy(
                r2_hbm.at[pl.ds(rr + 8, CRO_TAIL - 8), pl.ds(cs, SLAB)],
                rows_v[1].at[pl.ds(8, CRO_TAIL - 8)])
        selu_chunk(CRO_TAIL)
        pltpu.sync_copy(rows_v[0].at[pl.ds(0, 8)],
                        out_hbm.at[pl.ds(rr, 8), pl.ds(cs, SLAB)])

        @pl.when(t < NS - 1)
        def _():
            pltpu.sync_copy(rows_v[0].at[pl.ds(8, CRO_TAIL - 8)],
                            out_hbm.at[pl.ds(rr + 8, CRO_TAIL - 8),
                                       pl.ds(cs, SLAB)])
        plsc.subcore_barrier()


_sc_segsum256 = functools.partial(
    pl.kernel,
    out_type=jax.ShapeDtypeStruct((N, H), jnp.float32),
    mesh=plsc.VectorSubcoreMesh(core_axis_name="c", subcore_axis_name="s"),
    compiler_params=pltpu.CompilerParams(use_tc_tiling_on_sc=False),
    scratch_types=[
        pltpu.VMEM_SHARED((ACC_ROWS, SLAB), jnp.float32),
        pltpu.VMEM((IBLK,), jnp.int32),
        pltpu.VMEM((IBLK // CHUNK, CHUNK), jnp.int32),
        [pltpu.VMEM((CHUNK, SLAB), jnp.float32) for _ in range(2)],
        pltpu.VMEM((CR, SLAB), jnp.float32),
        [pltpu.SemaphoreType.DMA for _ in range(2)],
        [pltpu.SemaphoreType.DMA for _ in range(2)],
        pltpu.SemaphoreType.DMA,
    ],
)(_sc_segsum256_body)


_BR = 1000


def _tc_dense_body(part_ref, x_ref, w1rel_ref, w1root_ref, b1_ref,
                   w2rel_ref, w2root_ref, b2_ref, p2_ref, r2_ref):
    agg = part_ref[0] + part_ref[1]
    pre = (jnp.dot(agg, w1rel_ref[...], preferred_element_type=jnp.float32)
           + jnp.dot(x_ref[...], w1root_ref[...],
                     preferred_element_type=jnp.float32)
           + b1_ref[...])
    h = _selu(pre)
    p2 = jnp.dot(h, w2rel_ref[...], preferred_element_type=jnp.float32)
    r2_ref[...] = (jnp.dot(h, w2root_ref[...],
                           preferred_element_type=jnp.float32) + b2_ref[...])
    for j in range(NSLAB):
        p2_ref[j] = p2[:, j * SLAB:(j + 1) * SLAB]


def _tc_dense(part, x, w1rel16, w1root, b1, w2rel, w2root, b2):
    return pl.pallas_call(
        _tc_dense_body,
        grid=(N // _BR,),
        in_specs=[
            pl.BlockSpec((NC, _BR, 16), lambda i: (0, i, 0)),
            pl.BlockSpec((_BR, F_IN), lambda i: (i, 0)),
            pl.BlockSpec((16, H), lambda i: (0, 0)),
            pl.BlockSpec((F_IN, H), lambda i: (0, 0)),
            pl.BlockSpec((1, H), lambda i: (0, 0)),
            pl.BlockSpec((H, H), lambda i: (0, 0)),
            pl.BlockSpec((H, H), lambda i: (0, 0)),
            pl.BlockSpec((1, H), lambda i: (0, 0)),
        ],
        out_specs=[
            pl.BlockSpec((NSLAB, _BR, SLAB), lambda i: (0, i, 0)),
            pl.BlockSpec((_BR, H), lambda i: (i, 0)),
        ],
        out_shape=[
            jax.ShapeDtypeStruct((NSLAB, N, SLAB), jnp.float32),
            jax.ShapeDtypeStruct((N, H), jnp.float32),
        ],
    )(part, x, w1rel16, w1root, b1, w2rel, w2root, b2)


def kernel(x, edge_index, batch, W1_rel, W1_root, b1, W2_rel, W2_root, b2):
    src = edge_index[0]
    dst = edge_index[1]
    npad = EPAD - E
    src_p = jnp.concatenate([src, jnp.zeros((npad,), jnp.int32)])
    dst_p = jnp.concatenate([dst, jnp.full((npad,), N, jnp.int32)])
    dst2 = dst_p.reshape(EPAD // CHUNK, CHUNK)

    x16 = jnp.pad(x, ((0, 0), (0, 16 - F_IN)))
    w1rel16 = jnp.pad(W1_rel, ((0, 16 - F_IN), (0, 0)))
    z16 = jnp.zeros((CR, 16), jnp.float32)
    z32 = jnp.zeros((CR, SLAB), jnp.float32)

    part = _sc_segsum16(x16, src_p, dst2, z16)
    p2, r2 = _tc_dense(part, x, w1rel16, W1_root, b1.reshape(1, H),
                       W2_rel, W2_root, b2.reshape(1, H))
    return _sc_segsum256(p2.reshape(NSLAB * N, SLAB), src_p, dst2, z32, r2)


# speedup vs baseline: 1.0591x; 1.0502x over previous
import functools

import jax
import jax.numpy as jnp
from jax import lax
from jax.experimental import pallas as pl
from jax.experimental.pallas import tpu as pltpu
from jax.experimental.pallas import tpu_sc as plsc

N = 50000
E = 800000
F_IN = 14
H = 256

NC = 2
NS = 16
CHUNK = 256
IBLK = 14 * CHUNK
EPAD = ((E + NC * NS * CHUNK - 1) // (NC * NS * CHUNK)) * (NC * NS * CHUNK)
ACC_ROWS = 50048
RT = ACC_ROWS // NS
CR = 128
CR_TAIL = RT - (RT // CR) * CR
CRO = 256
CRO_TAIL = RT - 12 * CRO
NSLAB = 8
SLAB = H // NSLAB

_SELU_ALPHA = 1.6732632423543772
_SELU_SCALE = 1.0507009873554805


def _selu(x):
    return _SELU_SCALE * jnp.where(x > 0, x, _SELU_ALPHA * (jnp.exp(x) - 1.0))


def _sc_segsum16_body(x16_hbm, src_hbm, dst2_hbm, z16_hbm, part_hbm,
                      acc_sh, src_blk, dst_blk, rows_v, buf_v, sg, ss):
    c = lax.axis_index("c")
    t = lax.axis_index("s")
    tile_edges = EPAD // (NC * NS)
    base = (c * NS + t) * tile_edges

    pltpu.sync_copy(z16_hbm, buf_v)
    r0 = t * RT

    def zero_body(i, _):
        pltpu.sync_copy(buf_v, acc_sh.at[pl.ds(r0 + i * CR, CR)])
        return 0
    lax.fori_loop(0, RT // CR, zero_body, 0)
    pltpu.sync_copy(buf_v.at[pl.ds(0, CR_TAIL)],
                    acc_sh.at[pl.ds(r0 + (RT // CR) * CR, CR_TAIL)])
    plsc.subcore_barrier()

    def blk_body(g, _):
        e0 = base + g * IBLK
        pltpu.sync_copy(src_hbm.at[pl.ds(e0, IBLK)], src_blk)
        pltpu.sync_copy(dst2_hbm.at[pl.ds(e0 // CHUNK, IBLK // CHUNK)],
                        dst_blk)

        def edge_body(k, _):
            for b in range(2):
                ch = 2 * k + b

                @pl.when(g + k > 0)
                def _():
                    pltpu.make_async_copy(rows_v[b],
                                          acc_sh.at[dst_blk.at[ch]],
                                          ss[b]).wait()
                pltpu.async_copy(
                    x16_hbm.at[src_blk.at[pl.ds(ch * CHUNK, CHUNK)]],
                    rows_v[b], sg[b])
            for b in range(2):
                ch = 2 * k + b
                pltpu.make_async_copy(
                    x16_hbm.at[src_blk.at[pl.ds(ch * CHUNK, CHUNK)]],
                    rows_v[b], sg[b]).wait()
                pltpu.async_copy(rows_v[b], acc_sh.at[dst_blk.at[ch]],
                                 ss[b], add=True)
            return 0
        lax.fori_loop(0, IBLK // (2 * CHUNK), edge_body, 0)
        return 0
    lax.fori_loop(0, tile_edges // IBLK, blk_body, 0)
    for b in range(2):
        pltpu.make_async_copy(rows_v[b], acc_sh.at[dst_blk.at[0]],
                              ss[b]).wait()
    plsc.subcore_barrier()

    def out_body(i, _):
        rr = r0 + i * CR
        pltpu.sync_copy(acc_sh.at[pl.ds(rr, CR)], buf_v)
        pltpu.sync_copy(buf_v, part_hbm.at[c, pl.ds(rr, CR)])
        return 0
    lax.fori_loop(0, RT // CR, out_body, 0)
    rr = r0 + (RT // CR) * CR
    pltpu.sync_copy(acc_sh.at[pl.ds(rr, CR_TAIL)], buf_v.at[pl.ds(0, CR_TAIL)])
    pltpu.sync_copy(buf_v.at[pl.ds(0, CR_TAIL)],
                    part_hbm.at[c, pl.ds(rr, CR_TAIL)])


_sc_segsum16 = functools.partial(
    pl.kernel,
    out_type=jax.ShapeDtypeStruct((NC, ACC_ROWS, 16), jnp.float32),
    mesh=plsc.VectorSubcoreMesh(core_axis_name="c", subcore_axis_name="s"),
    compiler_params=pltpu.CompilerParams(use_tc_tiling_on_sc=False),
    scratch_types=[
        pltpu.VMEM_SHARED((ACC_ROWS, 16), jnp.float32),
        pltpu.VMEM((IBLK,), jnp.int32),
        pltpu.VMEM((IBLK // CHUNK, CHUNK), jnp.int32),
        [pltpu.VMEM((CHUNK, 16), jnp.float32) for _ in range(2)],
        pltpu.VMEM((CR, 16), jnp.float32),
        [pltpu.SemaphoreType.DMA for _ in range(2)],
        [pltpu.SemaphoreType.DMA for _ in range(2)],
    ],
)(_sc_segsum16_body)


def _sc_segsum256_body(p2_hbm, src_hbm, dst2_hbm, z32_hbm, agg_hbm,
                       acc_sh, src_blk, dst_blk, rows_v, buf_v, sg, ss, sz):
    c = lax.axis_index("c")
    t = lax.axis_index("s")
    tile_edges = EPAD // NS
    base = t * tile_edges
    r0 = t * RT

    for j in range(NSLAB // NC):
        s = c * (NSLAB // NC) + j
        s_off = s * N

        pltpu.sync_copy(z32_hbm, buf_v)

        def zero_start(i, _):
            pltpu.async_copy(buf_v, acc_sh.at[pl.ds(r0 + i * CR, CR)], sz)
            return 0
        lax.fori_loop(0, RT // CR, zero_start, 0)
        pltpu.sync_copy(buf_v.at[pl.ds(0, CR_TAIL)],
                        acc_sh.at[pl.ds(r0 + (RT // CR) * CR, CR_TAIL)])

        def zero_drain(i, _):
            pltpu.make_async_copy(buf_v,
                                  acc_sh.at[pl.ds(r0 + i * CR, CR)],
                                  sz).wait()
            return 0
        lax.fori_loop(0, RT // CR, zero_drain, 0)
        plsc.subcore_barrier()

        p2s = p2_hbm.at[pl.ds(s_off, N)]

        def blk_body(g, _):
            e0 = base + g * IBLK
            pltpu.sync_copy(src_hbm.at[pl.ds(e0, IBLK)], src_blk)
            pltpu.sync_copy(dst2_hbm.at[pl.ds(e0 // CHUNK, IBLK // CHUNK)],
                            dst_blk)

            def edge_body(k, _):
                for b in range(2):
                    ch = 2 * k + b

                    @pl.when(g + k > 0)
                    def _():
                        pltpu.make_async_copy(rows_v[b],
                                              acc_sh.at[dst_blk.at[ch]],
                                              ss[b]).wait()
                    pltpu.async_copy(
                        p2s.at[src_blk.at[pl.ds(ch * CHUNK, CHUNK)]],
                        rows_v[b], sg[b])
                for b in range(2):
                    ch = 2 * k + b
                    pltpu.make_async_copy(
                        p2s.at[src_blk.at[pl.ds(ch * CHUNK, CHUNK)]],
                        rows_v[b], sg[b]).wait()
                    pltpu.async_copy(rows_v[b], acc_sh.at[dst_blk.at[ch]],
                                     ss[b], add=True)
                return 0
            lax.fori_loop(0, IBLK // (2 * CHUNK), edge_body, 0)
            return 0
        lax.fori_loop(0, tile_edges // IBLK, blk_body, 0)
        for b in range(2):
            pltpu.make_async_copy(rows_v[b], acc_sh.at[dst_blk.at[0]],
                                  ss[b]).wait()
        plsc.subcore_barrier()

        def out_body(i, _):
            for b in range(2):
                rr = r0 + (2 * i + b) * CRO

                @pl.when(i > 0)
                def _():
                    pltpu.make_async_copy(
                        rows_v[b], agg_hbm.at[s, pl.ds(rr - 2 * CRO, CRO)],
                        ss[b]).wait()
                pltpu.async_copy(acc_sh.at[pl.ds(rr, CRO)], rows_v[b],
                                 sg[b])
            for b in range(2):
                rr = r0 + (2 * i + b) * CRO
                pltpu.make_async_copy(acc_sh.at[pl.ds(rr, CRO)], rows_v[b],
                                      sg[b]).wait()
                pltpu.async_copy(rows_v[b], agg_hbm.at[s, pl.ds(rr, CRO)],
                                 ss[b])
            return 0
        lax.fori_loop(0, RT // (2 * CRO), out_body, 0)
        rr = r0 + (RT // (2 * CRO)) * 2 * CRO
        pltpu.sync_copy(acc_sh.at[pl.ds(rr, CRO_TAIL)],
                        buf_v.at[pl.ds(0, CRO_TAIL)])
        pltpu.sync_copy(buf_v.at[pl.ds(0, CRO_TAIL)],
                        agg_hbm.at[s, pl.ds(rr, CRO_TAIL)])
        for b in range(2):
            pltpu.make_async_copy(rows_v[b], agg_hbm.at[s, pl.ds(r0, CRO)],
                                  ss[b]).wait()
        plsc.subcore_barrier()


_sc_segsum256 = functools.partial(
    pl.kernel,
    out_type=jax.ShapeDtypeStruct((NSLAB, ACC_ROWS, SLAB), jnp.float32),
    mesh=plsc.VectorSubcoreMesh(core_axis_name="c", subcore_axis_name="s"),
    compiler_params=pltpu.CompilerParams(use_tc_tiling_on_sc=False),
    scratch_types=[
        pltpu.VMEM_SHARED((ACC_ROWS, SLAB), jnp.float32),
        pltpu.VMEM((IBLK,), jnp.int32),
        pltpu.VMEM((IBLK // CHUNK, CHUNK), jnp.int32),
        [pltpu.VMEM((CHUNK, SLAB), jnp.float32) for _ in range(2)],
        pltpu.VMEM((CR, SLAB), jnp.float32),
        [pltpu.SemaphoreType.DMA for _ in range(2)],
        [pltpu.SemaphoreType.DMA for _ in range(2)],
        pltpu.SemaphoreType.DMA,
    ],
)(_sc_segsum256_body)


_BR = 1000


def _tc_dense_body(part_ref, x_ref, w1rel_ref, w1root_ref, b1_ref,
                   w2rel_ref, w2root_ref, b2_ref, p2_ref, r2_ref):
    agg = part_ref[0] + part_ref[1]
    pre = (jnp.dot(agg, w1rel_ref[...], preferred_element_type=jnp.float32)
           + jnp.dot(x_ref[...], w1root_ref[...],
                     preferred_element_type=jnp.float32)
           + b1_ref[...])
    h = _selu(pre)
    p2 = jnp.dot(h, w2rel_ref[...], preferred_element_type=jnp.float32)
    r2_ref[...] = (jnp.dot(h, w2root_ref[...],
                           preferred_element_type=jnp.float32) + b2_ref[...])
    for j in range(NSLAB):
        p2_ref[j] = p2[:, j * SLAB:(j + 1) * SLAB]


def _tc_dense(part, x, w1rel16, w1root, b1, w2rel, w2root, b2):
    return pl.pallas_call(
        _tc_dense_body,
        grid=(N // _BR,),
        in_specs=[
            pl.BlockSpec((NC, _BR, 16), lambda i: (0, i, 0)),
            pl.BlockSpec((_BR, F_IN), lambda i: (i, 0)),
            pl.BlockSpec((16, H), lambda i: (0, 0)),
            pl.BlockSpec((F_IN, H), lambda i: (0, 0)),
            pl.BlockSpec((1, H), lambda i: (0, 0)),
            pl.BlockSpec((H, H), lambda i: (0, 0)),
            pl.BlockSpec((H, H), lambda i: (0, 0)),
            pl.BlockSpec((1, H), lambda i: (0, 0)),
        ],
        out_specs=[
            pl.BlockSpec((NSLAB, _BR, SLAB), lambda i: (0, i, 0)),
            pl.BlockSpec((_BR, H), lambda i: (i, 0)),
        ],
        out_shape=[
            jax.ShapeDtypeStruct((NSLAB, N, SLAB), jnp.float32),
            jax.ShapeDtypeStruct((N, H), jnp.float32),
        ],
    )(part, x, w1rel16, w1root, b1, w2rel, w2root, b2)


def _tc_final_body(agg_ref, r2_ref, out_ref):
    parts = [agg_ref[j] for j in range(NSLAB)]
    out_ref[...] = _selu(jnp.concatenate(parts, axis=1) + r2_ref[...])


def _tc_final(agg2, r2):
    return pl.pallas_call(
        _tc_final_body,
        grid=(N // _BR,),
        in_specs=[
            pl.BlockSpec((NSLAB, _BR, SLAB), lambda i: (0, i, 0)),
            pl.BlockSpec((_BR, H), lambda i: (i, 0)),
        ],
        out_specs=pl.BlockSpec((_BR, H), lambda i: (i, 0)),
        out_shape=jax.ShapeDtypeStruct((N, H), jnp.float32),
    )(agg2, r2)


def kernel(x, edge_index, batch, W1_rel, W1_root, b1, W2_rel, W2_root, b2):
    src = edge_index[0]
    dst = edge_index[1]
    npad = EPAD - E
    src_p = jnp.concatenate([src, jnp.zeros((npad,), jnp.int32)])
    dst_p = jnp.concatenate([dst, jnp.full((npad,), N, jnp.int32)])
    dst2 = dst_p.reshape(EPAD // CHUNK, CHUNK)

    x16 = jnp.pad(x, ((0, 0), (0, 16 - F_IN)))
    w1rel16 = jnp.pad(W1_rel, ((0, 16 - F_IN), (0, 0)))
    z16 = jnp.zeros((CR, 16), jnp.float32)
    z32 = jnp.zeros((CR, SLAB), jnp.float32)

    part = _sc_segsum16(x16, src_p, dst2, z16)
    p2, r2 = _tc_dense(part, x, w1rel16, W1_root, b1.reshape(1, H),
                       W2_rel, W2_root, b2.reshape(1, H))
    agg2 = _sc_segsum256(p2.reshape(NSLAB * N, SLAB), src_p, dst2, z32)
    return _tc_final(agg2, r2)


# speedup vs baseline: 1.1552x; 1.0907x over previous
import functools

import jax
import jax.numpy as jnp
from jax import lax
from jax.experimental import pallas as pl
from jax.experimental.pallas import tpu as pltpu
from jax.experimental.pallas import tpu_sc as plsc

N = 50000
E = 800000
F_IN = 14
H = 256

NC = 2
NS = 16
CHUNK = 128
NCHB = 28
IBLK = NCHB * CHUNK
EPAD = ((E + NC * NS * CHUNK - 1) // (NC * NS * CHUNK)) * (NC * NS * CHUNK)
ACC_ROWS = 50048
RT = ACC_ROWS // NS
CR = 128
CR_TAIL = RT - (RT // CR) * CR
CRO = 128
CRO_TAIL = RT - 24 * CRO
NSLAB = 8
SLAB = H // NSLAB

_SELU_ALPHA = 1.6732632423543772
_SELU_SCALE = 1.0507009873554805


def _selu(x):
    return _SELU_SCALE * jnp.where(x > 0, x, _SELU_ALPHA * (jnp.exp(x) - 1.0))


def _sc_segsum16_body(x16_hbm, src_hbm, dst2_hbm, z16_hbm, part_hbm,
                      acc_sh, src_blk, dst_blk, rows_v, buf_v, sg, ss):
    c = lax.axis_index("c")
    t = lax.axis_index("s")
    tile_edges = EPAD // (NC * NS)
    base = (c * NS + t) * tile_edges

    pltpu.sync_copy(z16_hbm, buf_v)
    r0 = t * RT

    def zero_body(i, _):
        pltpu.sync_copy(buf_v, acc_sh.at[pl.ds(r0 + i * CR, CR)])
        return 0
    lax.fori_loop(0, RT // CR, zero_body, 0)
    pltpu.sync_copy(buf_v.at[pl.ds(0, CR_TAIL)],
                    acc_sh.at[pl.ds(r0 + (RT // CR) * CR, CR_TAIL)])
    plsc.subcore_barrier()

    def blk_body(g, _):
        e0 = base + g * IBLK
        pltpu.sync_copy(src_hbm.at[pl.ds(e0, IBLK)], src_blk)
        pltpu.sync_copy(dst2_hbm.at[pl.ds(e0 // CHUNK, NCHB)], dst_blk)
        for b in range(4):
            pltpu.async_copy(
                x16_hbm.at[src_blk.at[pl.ds(b * CHUNK, CHUNK)]],
                rows_v[b], sg[b])

        def grp_body(k, _):
            for b in range(4):
                ch = 4 * k + b
                pltpu.make_async_copy(
                    x16_hbm.at[src_blk.at[pl.ds(ch * CHUNK, CHUNK)]],
                    rows_v[b], sg[b]).wait()
                pltpu.async_copy(rows_v[b], acc_sh.at[dst_blk.at[ch]],
                                 ss[b], add=True)
            for b in range(4):
                ch = 4 * (k + 1) + b
                pltpu.make_async_copy(rows_v[b], acc_sh.at[dst_blk.at[ch]],
                                      ss[b]).wait()
                pltpu.async_copy(
                    x16_hbm.at[src_blk.at[pl.ds(ch * CHUNK, CHUNK)]],
                    rows_v[b], sg[b])
            return 0
        lax.fori_loop(0, NCHB // 4 - 1, grp_body, 0)
        for b in range(4):
            ch = NCHB - 4 + b
            pltpu.make_async_copy(
                x16_hbm.at[src_blk.at[pl.ds(ch * CHUNK, CHUNK)]],
                rows_v[b], sg[b]).wait()
            pltpu.async_copy(rows_v[b], acc_sh.at[dst_blk.at[ch]],
                             ss[b], add=True)
        for b in range(4):
            pltpu.make_async_copy(rows_v[b], acc_sh.at[dst_blk.at[0]],
                                  ss[b]).wait()
        return 0
    lax.fori_loop(0, tile_edges // IBLK, blk_body, 0)
    plsc.subcore_barrier()

    def out_body(i, _):
        rr = r0 + i * CR
        pltpu.sync_copy(acc_sh.at[pl.ds(rr, CR)], buf_v)
        pltpu.sync_copy(buf_v, part_hbm.at[c, pl.ds(rr, CR)])
        return 0
    lax.fori_loop(0, RT // CR, out_body, 0)
    rr = r0 + (RT // CR) * CR
    pltpu.sync_copy(acc_sh.at[pl.ds(rr, CR_TAIL)], buf_v.at[pl.ds(0, CR_TAIL)])
    pltpu.sync_copy(buf_v.at[pl.ds(0, CR_TAIL)],
                    part_hbm.at[c, pl.ds(rr, CR_TAIL)])


_sc_segsum16 = functools.partial(
    pl.kernel,
    out_type=jax.ShapeDtypeStruct((NC, ACC_ROWS, 16), jnp.float32),
    mesh=plsc.VectorSubcoreMesh(core_axis_name="c", subcore_axis_name="s"),
    compiler_params=pltpu.CompilerParams(use_tc_tiling_on_sc=False),
    scratch_types=[
        pltpu.VMEM_SHARED((ACC_ROWS, 16), jnp.float32),
        pltpu.VMEM((IBLK,), jnp.int32),
        pltpu.VMEM((NCHB, CHUNK), jnp.int32),
        [pltpu.VMEM((CHUNK, 16), jnp.float32) for _ in range(4)],
        pltpu.VMEM((CR, 16), jnp.float32),
        [pltpu.SemaphoreType.DMA for _ in range(4)],
        [pltpu.SemaphoreType.DMA for _ in range(4)],
    ],
)(_sc_segsum16_body)


def _sc_segsum256_body(p2_hbm, src_hbm, dst2_hbm, z32_hbm, agg_hbm,
                       acc_sh, src_blk, dst_blk, rows_v, buf_v, sg, ss, sz):
    c = lax.axis_index("c")
    t = lax.axis_index("s")
    tile_edges = EPAD // NS
    base = t * tile_edges
    r0 = t * RT

    for j in range(NSLAB // NC):
        s = c * (NSLAB // NC) + j
        s_off = s * N

        pltpu.sync_copy(z32_hbm, buf_v)

        def zero_start(i, _):
            pltpu.async_copy(buf_v, acc_sh.at[pl.ds(r0 + i * CR, CR)], sz)
            return 0
        lax.fori_loop(0, RT // CR, zero_start, 0)
        pltpu.sync_copy(buf_v.at[pl.ds(0, CR_TAIL)],
                        acc_sh.at[pl.ds(r0 + (RT // CR) * CR, CR_TAIL)])

        def zero_drain(i, _):
            pltpu.make_async_copy(buf_v,
                                  acc_sh.at[pl.ds(r0 + i * CR, CR)],
                                  sz).wait()
            return 0
        lax.fori_loop(0, RT // CR, zero_drain, 0)
        plsc.subcore_barrier()

        p2s = p2_hbm.at[pl.ds(s_off, N)]

        def blk_body(g, _):
            e0 = base + g * IBLK
            pltpu.sync_copy(src_hbm.at[pl.ds(e0, IBLK)], src_blk)
            pltpu.sync_copy(dst2_hbm.at[pl.ds(e0 // CHUNK, NCHB)], dst_blk)
            for b in range(4):
                pltpu.async_copy(
                    p2s.at[src_blk.at[pl.ds(b * CHUNK, CHUNK)]],
                    rows_v[b], sg[b])

            def grp_body(k, _):
                for b in range(4):
                    ch = 4 * k + b
                    pltpu.make_async_copy(
                        p2s.at[src_blk.at[pl.ds(ch * CHUNK, CHUNK)]],
                        rows_v[b], sg[b]).wait()
                    pltpu.async_copy(rows_v[b], acc_sh.at[dst_blk.at[ch]],
                                     ss[b], add=True)
                for b in range(4):
                    ch = 4 * (k + 1) + b
                    pltpu.make_async_copy(rows_v[b],
                                          acc_sh.at[dst_blk.at[ch]],
                                          ss[b]).wait()
                    pltpu.async_copy(
                        p2s.at[src_blk.at[pl.ds(ch * CHUNK, CHUNK)]],
                        rows_v[b], sg[b])
                return 0
            lax.fori_loop(0, NCHB // 4 - 1, grp_body, 0)
            for b in range(4):
                ch = NCHB - 4 + b
                pltpu.make_async_copy(
                    p2s.at[src_blk.at[pl.ds(ch * CHUNK, CHUNK)]],
                    rows_v[b], sg[b]).wait()
                pltpu.async_copy(rows_v[b], acc_sh.at[dst_blk.at[ch]],
                                 ss[b], add=True)
            for b in range(4):
                pltpu.make_async_copy(rows_v[b], acc_sh.at[dst_blk.at[0]],
                                      ss[b]).wait()
            return 0
        lax.fori_loop(0, tile_edges // IBLK, blk_body, 0)
        plsc.subcore_barrier()

        def out_body(i, _):
            for b in range(2):
                rr = r0 + (2 * i + b) * CRO

                @pl.when(i > 0)
                def _():
                    pltpu.make_async_copy(
                        rows_v[b], agg_hbm.at[s, pl.ds(rr - 2 * CRO, CRO)],
                        ss[b]).wait()
                pltpu.async_copy(acc_sh.at[pl.ds(rr, CRO)], rows_v[b],
                                 sg[b])
            for b in range(2):
                rr = r0 + (2 * i + b) * CRO
                pltpu.make_async_copy(acc_sh.at[pl.ds(rr, CRO)], rows_v[b],
                                      sg[b]).wait()
                pltpu.async_copy(rows_v[b], agg_hbm.at[s, pl.ds(rr, CRO)],
                                 ss[b])
            return 0
        lax.fori_loop(0, RT // (2 * CRO), out_body, 0)
        rr = r0 + (RT // (2 * CRO)) * 2 * CRO
        pltpu.sync_copy(acc_sh.at[pl.ds(rr, CRO_TAIL)],
                        buf_v.at[pl.ds(0, CRO_TAIL)])
        pltpu.sync_copy(buf_v.at[pl.ds(0, CRO_TAIL)],
                        agg_hbm.at[s, pl.ds(rr, CRO_TAIL)])
        for b in range(2):
            pltpu.make_async_copy(rows_v[b], agg_hbm.at[s, pl.ds(r0, CRO)],
                                  ss[b]).wait()
        plsc.subcore_barrier()


_sc_segsum256 = functools.partial(
    pl.kernel,
    out_type=jax.ShapeDtypeStruct((NSLAB, ACC_ROWS, SLAB), jnp.float32),
    mesh=plsc.VectorSubcoreMesh(core_axis_name="c", subcore_axis_name="s"),
    compiler_params=pltpu.CompilerParams(use_tc_tiling_on_sc=False),
    scratch_types=[
        pltpu.VMEM_SHARED((ACC_ROWS, SLAB), jnp.float32),
        pltpu.VMEM((IBLK,), jnp.int32),
        pltpu.VMEM((NCHB, CHUNK), jnp.int32),
        [pltpu.VMEM((CHUNK, SLAB), jnp.float32) for _ in range(4)],
        pltpu.VMEM((CR, SLAB), jnp.float32),
        [pltpu.SemaphoreType.DMA for _ in range(4)],
        [pltpu.SemaphoreType.DMA for _ in range(4)],
        pltpu.SemaphoreType.DMA,
    ],
)(_sc_segsum256_body)


_BR = 1000


def _tc_dense_body(part_ref, x_ref, w1rel_ref, w1root_ref, b1_ref,
                   w2rel_ref, w2root_ref, b2_ref, p2_ref, r2_ref):
    agg = part_ref[0] + part_ref[1]
    pre = (jnp.dot(agg, w1rel_ref[...], preferred_element_type=jnp.float32)
           + jnp.dot(x_ref[...], w1root_ref[...],
                     preferred_element_type=jnp.float32)
           + b1_ref[...])
    h = _selu(pre)
    p2 = jnp.dot(h, w2rel_ref[...], preferred_element_type=jnp.float32)
    r2_ref[...] = (jnp.dot(h, w2root_ref[...],
                           preferred_element_type=jnp.float32) + b2_ref[...])
    for j in range(NSLAB):
        p2_ref[j] = p2[:, j * SLAB:(j + 1) * SLAB]


def _tc_dense(part, x, w1rel16, w1root, b1, w2rel, w2root, b2):
    return pl.pallas_call(
        _tc_dense_body,
        grid=(N // _BR,),
        in_specs=[
            pl.BlockSpec((NC, _BR, 16), lambda i: (0, i, 0)),
            pl.BlockSpec((_BR, F_IN), lambda i: (i, 0)),
            pl.BlockSpec((16, H), lambda i: (0, 0)),
            pl.BlockSpec((F_IN, H), lambda i: (0, 0)),
            pl.BlockSpec((1, H), lambda i: (0, 0)),
            pl.BlockSpec((H, H), lambda i: (0, 0)),
            pl.BlockSpec((H, H), lambda i: (0, 0)),
            pl.BlockSpec((1, H), lambda i: (0, 0)),
        ],
        out_specs=[
            pl.BlockSpec((NSLAB, _BR, SLAB), lambda i: (0, i, 0)),
            pl.BlockSpec((_BR, H), lambda i: (i, 0)),
        ],
        out_shape=[
            jax.ShapeDtypeStruct((NSLAB, N, SLAB), jnp.float32),
            jax.ShapeDtypeStruct((N, H), jnp.float32),
        ],
    )(part, x, w1rel16, w1root, b1, w2rel, w2root, b2)


def _tc_final_body(agg_ref, r2_ref, out_ref):
    parts = [agg_ref[j] for j in range(NSLAB)]
    out_ref[...] = _selu(jnp.concatenate(parts, axis=1) + r2_ref[...])


def _tc_final(agg2, r2):
    return pl.pallas_call(
        _tc_final_body,
        grid=(N // _BR,),
        in_specs=[
            pl.BlockSpec((NSLAB, _BR, SLAB), lambda i: (0, i, 0)),
            pl.BlockSpec((_BR, H), lambda i: (i, 0)),
        ],
        out_specs=pl.BlockSpec((_BR, H), lambda i: (i, 0)),
        out_shape=jax.ShapeDtypeStruct((N, H), jnp.float32),
    )(agg2, r2)


def kernel(x, edge_index, batch, W1_rel, W1_root, b1, W2_rel, W2_root, b2):
    src = edge_index[0]
    dst = edge_index[1]
    npad = EPAD - E
    src_p = jnp.concatenate([src, jnp.zeros((npad,), jnp.int32)])
    dst_p = jnp.concatenate([dst, jnp.full((npad,), N, jnp.int32)])
    dst2 = dst_p.reshape(EPAD // CHUNK, CHUNK)

    x16 = jnp.pad(x, ((0, 0), (0, 16 - F_IN)))
    w1rel16 = jnp.pad(W1_rel, ((0, 16 - F_IN), (0, 0)))
    z16 = jnp.zeros((CR, 16), jnp.float32)
    z32 = jnp.zeros((CR, SLAB), jnp.float32)

    part = _sc_segsum16(x16, src_p, dst2, z16)
    p2, r2 = _tc_dense(part, x, w1rel16, W1_root, b1.reshape(1, H),
                       W2_rel, W2_root, b2.reshape(1, H))
    agg2 = _sc_segsum256(p2.reshape(NSLAB * N, SLAB), src_p, dst2, z32)
    return _tc_final(agg2, r2)
